# Initial kernel scaffold; baseline (speedup 1.0000x reference)
import jax
import jax.numpy as jnp
from jax.experimental import pallas as pl


def kernel(x, edge_index, W1_l, b1_l, W1_r, W2_l, b2_l, W2_r):
    raise NotImplementedError("write your pallas kernel here")


# speedup vs baseline: 6.2387x; 6.2387x over previous
import functools

import jax
import jax.numpy as jnp
from jax import lax
from jax.experimental import pallas as pl
from jax.experimental.pallas import tpu as pltpu
from jax.experimental.pallas import tpu_sc as plsc

N = 10000
E = 320000
D_IN = 128
D_HID = 256

NC = 2
NS = 16
NW = NC * NS

CH = 128
NCH = E // CH
SS = 8
SST = (NCH + SS - 1) // SS
SPW = SST // NW
EXTRA = SST % NW
NCH_PAD = SST * SS

NP = 10240
RPT = NP // NS

_mesh = plsc.VectorSubcoreMesh(core_axis_name="c", subcore_axis_name="s")


def _worker_id():
    c = lax.axis_index("c")
    s = lax.axis_index("s")
    return c, s, s * NC + c


SPW1 = SST // NS
EXTRA1 = SST % NS


def _zero_rows(rows):
    def body(i, _):
        rows[i // 8, pl.ds((i % 8) * 16, 16)] = jnp.zeros((16,), jnp.float32)
        return 0
    lax.fori_loop(0, CH * 8, body, 0)


def _fill_ones(ones):
    def body(i, _):
        ones[i // 8, pl.ds((i % 8) * 16, 16)] = jnp.ones((16,), jnp.float32)
        return 0
    lax.fori_loop(0, CH * 8, body, 0)


def _zero_acc(rows, acc, t):
    for k in range(RPT // CH):
        pltpu.sync_copy(rows, acc.at[pl.ds(t * RPT + k * CH, CH)])


def _edge_loop(tab_hbm, srcp_hbm, dstp_hbm, src_idx, dst_idx, rows, acc, sem,
               w, ones=None, cnt_acc=None):
    start_ss = w * SPW + jnp.minimum(w, EXTRA)
    n_ss = jnp.where(w < EXTRA, SPW + 1, SPW)

    def ss_body(si, _):
        ssi = start_ss + si
        pltpu.sync_copy(srcp_hbm.at[ssi], src_idx)
        pltpu.sync_copy(dstp_hbm.at[ssi], dst_idx)
        k = jnp.minimum(NCH - ssi * SS, SS)

        def ch_body(j, _):
            pltpu.async_copy(tab_hbm.at[src_idx.at[j]], rows, sem).wait()
            pltpu.sync_copy(rows, acc.at[dst_idx.at[j]], add=True)
            if cnt_acc is not None:
                pltpu.sync_copy(ones, cnt_acc.at[dst_idx.at[j]], add=True)
            return 0

        lax.fori_loop(0, k, ch_body, 0)
        return 0

    lax.fori_loop(0, n_ss, ss_body, 0)


def _write_out(acc, out_hbm, base, t):
    pltpu.sync_copy(acc.at[pl.ds(t * RPT, RPT)],
                    out_hbm.at[pl.ds(base + t * RPT, RPT)])


def _cnt_loop(dstp_hbm, dst_idx, rows, acc, w):
    start_ss = w * SPW + jnp.minimum(w, EXTRA)
    n_ss = jnp.where(w < EXTRA, SPW + 1, SPW)

    def ss_body(si, _):
        ssi = start_ss + si
        pltpu.sync_copy(dstp_hbm.at[ssi], dst_idx)
        k = jnp.minimum(NCH - ssi * SS, SS)

        def ch_body(j, _):
            pltpu.sync_copy(rows, acc.at[dst_idx.at[j]], add=True)
            return 0

        lax.fori_loop(0, k, ch_body, 0)
        return 0

    lax.fori_loop(0, n_ss, ss_body, 0)


@functools.partial(
    pl.kernel,
    mesh=_mesh,
    out_type=[
        jax.ShapeDtypeStruct((NC * NP, D_IN), jnp.float32),
        jax.ShapeDtypeStruct((NC * NP, D_IN), jnp.float32),
    ],
    scratch_types=[
        pltpu.VMEM((SS, CH), jnp.int32),
        pltpu.VMEM((SS, CH), jnp.int32),
        pltpu.VMEM((CH, D_IN), jnp.float32),
        pltpu.VMEM_SHARED((NP, D_IN), jnp.float32),
        pltpu.SemaphoreType.DMA,
    ],
)
def _sc_layer1(x_hbm, srcp_hbm, dstp_hbm, agg_out, cnt_out,
               src_idx, dst_idx, rows, acc, sem):
    c, t, w = _worker_id()
    _zero_rows(rows)
    _zero_acc(rows, acc, t)
    plsc.subcore_barrier()
    _edge_loop(x_hbm, srcp_hbm, dstp_hbm, src_idx, dst_idx, rows, acc, sem, w)
    plsc.subcore_barrier()
    _write_out(acc, agg_out, c * NP, t)
    plsc.subcore_barrier()
    _zero_rows(rows)
    _zero_acc(rows, acc, t)
    _fill_ones(rows)
    plsc.subcore_barrier()
    _cnt_loop(dstp_hbm, dst_idx, rows, acc, w)
    plsc.subcore_barrier()
    _write_out(acc, cnt_out, c * NP, t)


@functools.partial(
    pl.kernel,
    mesh=_mesh,
    out_type=jax.ShapeDtypeStruct((2 * NC * NP, D_IN), jnp.float32),
    scratch_types=[
        pltpu.VMEM((SS, CH), jnp.int32),
        pltpu.VMEM((SS, CH), jnp.int32),
        pltpu.VMEM((CH, D_IN), jnp.float32),
        pltpu.VMEM_SHARED((NP, D_IN), jnp.float32),
        pltpu.SemaphoreType.DMA,
    ],
)
def _sc_layer2(h0_hbm, h1_hbm, srcp_hbm, dstp_hbm, agg_out,
               src_idx, dst_idx, rows, acc, sem):
    c, t, w = _worker_id()
    for half, tab in ((0, h0_hbm), (1, h1_hbm)):
        _zero_rows(rows)
        _zero_acc(rows, acc, t)
        plsc.subcore_barrier()
        _edge_loop(tab, srcp_hbm, dstp_hbm, src_idx, dst_idx, rows, acc, sem, w)
        plsc.subcore_barrier()
        _write_out(acc, agg_out, (half * NC + c) * NP, t)
        plsc.subcore_barrier()


def _dg_t(a, w):
    return lax.dot_general(a, w, (((1,), (1,)), ((), ())),
                           preferred_element_type=jnp.float32)


M_BLK = 1000
M_GRID = N // M_BLK


def _tc1_body(agg_ref, cnt_ref, x_ref, wl_ref, b_ref, wr_ref, o_ref):
    cnt = jnp.maximum(cnt_ref[0, :, 0:1] + cnt_ref[1, :, 0:1], 1.0)
    mean = (agg_ref[0] + agg_ref[1]) / cnt
    out = _dg_t(mean, wl_ref[...]) + _dg_t(x_ref[...], wr_ref[...])
    o_ref[0] = jnp.maximum(out + b_ref[0], 0.0)


def _tc_dense1(agg, cnt, x, W1_l, b1, W1_r):
    return pl.pallas_call(
        _tc1_body,
        grid=(M_GRID, 2),
        in_specs=[
            pl.BlockSpec((2, M_BLK, D_IN), lambda i, j: (0, i, 0)),
            pl.BlockSpec((2, M_BLK, D_IN), lambda i, j: (0, i, 0)),
            pl.BlockSpec((M_BLK, D_IN), lambda i, j: (i, 0)),
            pl.BlockSpec((128, D_IN), lambda i, j: (j, 0)),
            pl.BlockSpec((1, 1, 128), lambda i, j: (j, 0, 0)),
            pl.BlockSpec((128, D_IN), lambda i, j: (j, 0)),
        ],
        out_specs=pl.BlockSpec((1, M_BLK, 128), lambda i, j: (j, i, 0)),
        out_shape=jax.ShapeDtypeStruct((2, N, 128), jnp.float32),
    )(agg, cnt, x, W1_l, b1, W1_r)


def _tc2_body(agg_ref, cnt_ref, h_ref, wl_ref, b_ref, wr_ref, o_ref):
    cnt = jnp.maximum(cnt_ref[0, :, 0:1] + cnt_ref[1, :, 0:1], 1.0)
    m0 = (agg_ref[0, 0] + agg_ref[0, 1]) / cnt
    m1 = (agg_ref[1, 0] + agg_ref[1, 1]) / cnt
    out = (_dg_t(m0, wl_ref[:, 0:128]) + _dg_t(m1, wl_ref[:, 128:256])
           + _dg_t(h_ref[0], wr_ref[:, 0:128]) + _dg_t(h_ref[1], wr_ref[:, 128:256]))
    o_ref[...] = out + b_ref[...]


def _tc_dense2(agg2, cnt, h, W2_l, b2, W2_r):
    return pl.pallas_call(
        _tc2_body,
        grid=(M_GRID,),
        in_specs=[
            pl.BlockSpec((2, 2, M_BLK, D_IN), lambda i: (0, 0, i, 0)),
            pl.BlockSpec((2, M_BLK, D_IN), lambda i: (0, i, 0)),
            pl.BlockSpec((2, M_BLK, 128), lambda i: (0, i, 0)),
            pl.BlockSpec((D_HID, D_HID), lambda i: (0, 0)),
            pl.BlockSpec((1, D_HID), lambda i: (0, 0)),
            pl.BlockSpec((D_HID, D_HID), lambda i: (0, 0)),
        ],
        out_specs=pl.BlockSpec((M_BLK, D_HID), lambda i: (i, 0)),
        out_shape=jax.ShapeDtypeStruct((N, D_HID), jnp.float32),
    )(agg2, cnt, h, W2_l, b2, W2_r)


def kernel(x, edge_index, W1_l, b1_l, W1_r, W2_l, b2_l, W2_r):
    src = edge_index[0].astype(jnp.int32).reshape(NCH, CH)
    dst = edge_index[1].astype(jnp.int32).reshape(NCH, CH)
    srcp = jnp.pad(src, ((0, NCH_PAD - NCH), (0, 0))).reshape(SST, SS, CH)
    dstp = jnp.pad(dst, ((0, NCH_PAD - NCH), (0, 0))).reshape(SST, SS, CH)

    agg1_f, cnt_f = _sc_layer1(x, srcp, dstp)
    agg1 = agg1_f.reshape(NC, NP, D_IN)
    cnt = cnt_f.reshape(NC, NP, D_IN)

    h = _tc_dense1(agg1, cnt, x, W1_l, b1_l.reshape(2, 1, 128), W1_r)

    agg2_f = _sc_layer2(h[0], h[1], srcp, dstp)
    agg2 = agg2_f.reshape(2, NC, NP, D_IN)

    return _tc_dense2(agg2, cnt, h, W2_l, b2_l.reshape(1, D_HID), W2_r)


# speedup vs baseline: 7.7029x; 1.2347x over previous
import functools

import jax
import jax.numpy as jnp
from jax import lax
from jax.experimental import pallas as pl
from jax.experimental.pallas import tpu as pltpu
from jax.experimental.pallas import tpu_sc as plsc

N = 10000
E = 320000
D_IN = 128
D_HID = 256

NC = 2
NS = 16
NW = NC * NS

CH = 128
NCH = E // CH
SS = 16
SST = (NCH + SS - 1) // SS
SPW = SST // NW
EXTRA = SST % NW
NCH_PAD = SST * SS

NP = 10240
RPT = NP // NS

_mesh = plsc.VectorSubcoreMesh(core_axis_name="c", subcore_axis_name="s")


def _worker_id():
    c = lax.axis_index("c")
    s = lax.axis_index("s")
    return c, s, s * NC + c


SPW1 = SST // NS
EXTRA1 = SST % NS


def _zero_rows(rows):
    def body(i, _):
        rows[i // 8, pl.ds((i % 8) * 16, 16)] = jnp.zeros((16,), jnp.float32)
        return 0
    lax.fori_loop(0, CH * 8, body, 0)


def _fill_ones(ones):
    def body(i, _):
        ones[i // 8, pl.ds((i % 8) * 16, 16)] = jnp.ones((16,), jnp.float32)
        return 0
    lax.fori_loop(0, CH * 8, body, 0)


def _zero_acc(rows, acc, t):
    for k in range(RPT // CH):
        pltpu.sync_copy(rows, acc.at[pl.ds(t * RPT + k * CH, CH)])


def _edge_loop(tab_hbm, srcp_hbm, dstp_hbm, src_idx, dst_idx,
               rows_a, rows_b, acc, sem_a, sem_b, w):
    start_ss = w * SPW + jnp.minimum(w, EXTRA)
    n_ss = jnp.where(w < EXTRA, SPW + 1, SPW)

    def _wait(buf, sem):
        pltpu.make_async_copy(tab_hbm.at[src_idx.at[0]], buf, sem).wait()

    def ss_body(si, _):
        ssi = start_ss + si
        pltpu.sync_copy(srcp_hbm.at[ssi], src_idx)
        pltpu.sync_copy(dstp_hbm.at[ssi], dst_idx)
        pltpu.async_copy(tab_hbm.at[src_idx.at[0]], rows_a, sem_a)

        def pair_body(jj, _):
            j0 = 2 * jj
            _wait(rows_a, sem_a)
            pltpu.async_copy(tab_hbm.at[src_idx.at[j0 + 1]], rows_b, sem_b)
            pltpu.sync_copy(rows_a, acc.at[dst_idx.at[j0]], add=True)
            _wait(rows_b, sem_b)
            nxt = jnp.minimum(j0 + 2, SS - 1)
            pltpu.async_copy(tab_hbm.at[src_idx.at[nxt]], rows_a, sem_a)
            pltpu.sync_copy(rows_b, acc.at[dst_idx.at[j0 + 1]], add=True)
            return 0

        lax.fori_loop(0, SS // 2, pair_body, 0)
        _wait(rows_a, sem_a)
        return 0

    lax.fori_loop(0, n_ss, ss_body, 0)


def _write_out(acc, out_hbm, base, t):
    pltpu.sync_copy(acc.at[pl.ds(t * RPT, RPT)],
                    out_hbm.at[pl.ds(base + t * RPT, RPT)])


def _cnt_loop(dstp_hbm, dst_idx, rows, acc, w):
    start_ss = w * SPW + jnp.minimum(w, EXTRA)
    n_ss = jnp.where(w < EXTRA, SPW + 1, SPW)

    def ss_body(si, _):
        ssi = start_ss + si
        pltpu.sync_copy(dstp_hbm.at[ssi], dst_idx)

        def ch_body(j, _):
            pltpu.sync_copy(rows, acc.at[dst_idx.at[j]], add=True)
            return 0

        lax.fori_loop(0, SS, ch_body, 0)
        return 0

    lax.fori_loop(0, n_ss, ss_body, 0)


@functools.partial(
    pl.kernel,
    mesh=_mesh,
    out_type=[
        jax.ShapeDtypeStruct((NC * NP, D_IN), jnp.float32),
        jax.ShapeDtypeStruct((NC * NP, D_IN), jnp.float32),
    ],
    scratch_types=[
        pltpu.VMEM((SS, CH), jnp.int32),
        pltpu.VMEM((SS, CH), jnp.int32),
        pltpu.VMEM((CH, D_IN), jnp.float32),
        pltpu.VMEM((CH, D_IN), jnp.float32),
        pltpu.VMEM_SHARED((NP, D_IN), jnp.float32),
        pltpu.SemaphoreType.DMA,
        pltpu.SemaphoreType.DMA,
    ],
)
def _sc_layer1(x_hbm, srcp_hbm, dstp_hbm, agg_out, cnt_out,
               src_idx, dst_idx, rows_a, rows_b, acc, sem_a, sem_b):
    c, t, w = _worker_id()
    _zero_rows(rows_a)
    _zero_acc(rows_a, acc, t)
    plsc.subcore_barrier()
    _edge_loop(x_hbm, srcp_hbm, dstp_hbm, src_idx, dst_idx,
               rows_a, rows_b, acc, sem_a, sem_b, w)
    plsc.subcore_barrier()
    _write_out(acc, agg_out, c * NP, t)
    plsc.subcore_barrier()
    _zero_rows(rows_a)
    _zero_acc(rows_a, acc, t)
    _fill_ones(rows_a)
    plsc.subcore_barrier()
    _cnt_loop(dstp_hbm, dst_idx, rows_a, acc, w)
    plsc.subcore_barrier()
    _write_out(acc, cnt_out, c * NP, t)


@functools.partial(
    pl.kernel,
    mesh=_mesh,
    out_type=jax.ShapeDtypeStruct((2 * NC * NP, D_IN), jnp.float32),
    scratch_types=[
        pltpu.VMEM((SS, CH), jnp.int32),
        pltpu.VMEM((SS, CH), jnp.int32),
        pltpu.VMEM((CH, D_IN), jnp.float32),
        pltpu.VMEM((CH, D_IN), jnp.float32),
        pltpu.VMEM_SHARED((NP, D_IN), jnp.float32),
        pltpu.SemaphoreType.DMA,
        pltpu.SemaphoreType.DMA,
    ],
)
def _sc_layer2(h0_hbm, h1_hbm, srcp_hbm, dstp_hbm, agg_out,
               src_idx, dst_idx, rows_a, rows_b, acc, sem_a, sem_b):
    c, t, w = _worker_id()
    for half, tab in ((0, h0_hbm), (1, h1_hbm)):
        _zero_rows(rows_a)
        _zero_acc(rows_a, acc, t)
        plsc.subcore_barrier()
        _edge_loop(tab, srcp_hbm, dstp_hbm, src_idx, dst_idx,
                   rows_a, rows_b, acc, sem_a, sem_b, w)
        plsc.subcore_barrier()
        _write_out(acc, agg_out, (half * NC + c) * NP, t)
        plsc.subcore_barrier()


def _dg_t(a, w):
    return lax.dot_general(a, w, (((1,), (1,)), ((), ())),
                           preferred_element_type=jnp.float32)


M_BLK = 1000
M_GRID = N // M_BLK


def _tc1_body(agg_ref, cnt_ref, x_ref, wl_ref, b_ref, wr_ref, o_ref):
    cnt = jnp.maximum(cnt_ref[0, :, 0:1] + cnt_ref[1, :, 0:1], 1.0)
    mean = (agg_ref[0] + agg_ref[1]) / cnt
    out = _dg_t(mean, wl_ref[...]) + _dg_t(x_ref[...], wr_ref[...])
    o_ref[0] = jnp.maximum(out + b_ref[0], 0.0)


def _tc_dense1(agg, cnt, x, W1_l, b1, W1_r):
    return pl.pallas_call(
        _tc1_body,
        grid=(M_GRID, 2),
        in_specs=[
            pl.BlockSpec((2, M_BLK, D_IN), lambda i, j: (0, i, 0)),
            pl.BlockSpec((2, M_BLK, D_IN), lambda i, j: (0, i, 0)),
            pl.BlockSpec((M_BLK, D_IN), lambda i, j: (i, 0)),
            pl.BlockSpec((128, D_IN), lambda i, j: (j, 0)),
            pl.BlockSpec((1, 1, 128), lambda i, j: (j, 0, 0)),
            pl.BlockSpec((128, D_IN), lambda i, j: (j, 0)),
        ],
        out_specs=pl.BlockSpec((1, M_BLK, 128), lambda i, j: (j, i, 0)),
        out_shape=jax.ShapeDtypeStruct((2, N, 128), jnp.float32),
    )(agg, cnt, x, W1_l, b1, W1_r)


def _tc2_body(agg_ref, cnt_ref, h_ref, wl_ref, b_ref, wr_ref, o_ref):
    cnt = jnp.maximum(cnt_ref[0, :, 0:1] + cnt_ref[1, :, 0:1], 1.0)
    m0 = (agg_ref[0, 0] + agg_ref[0, 1]) / cnt
    m1 = (agg_ref[1, 0] + agg_ref[1, 1]) / cnt
    out = (_dg_t(m0, wl_ref[:, 0:128]) + _dg_t(m1, wl_ref[:, 128:256])
           + _dg_t(h_ref[0], wr_ref[:, 0:128]) + _dg_t(h_ref[1], wr_ref[:, 128:256]))
    o_ref[...] = out + b_ref[...]


def _tc_dense2(agg2, cnt, h, W2_l, b2, W2_r):
    return pl.pallas_call(
        _tc2_body,
        grid=(M_GRID,),
        in_specs=[
            pl.BlockSpec((2, 2, M_BLK, D_IN), lambda i: (0, 0, i, 0)),
            pl.BlockSpec((2, M_BLK, D_IN), lambda i: (0, i, 0)),
            pl.BlockSpec((2, M_BLK, 128), lambda i: (0, i, 0)),
            pl.BlockSpec((D_HID, D_HID), lambda i: (0, 0)),
            pl.BlockSpec((1, D_HID), lambda i: (0, 0)),
            pl.BlockSpec((D_HID, D_HID), lambda i: (0, 0)),
        ],
        out_specs=pl.BlockSpec((M_BLK, D_HID), lambda i: (i, 0)),
        out_shape=jax.ShapeDtypeStruct((N, D_HID), jnp.float32),
    )(agg2, cnt, h, W2_l, b2, W2_r)


def kernel(x, edge_index, W1_l, b1_l, W1_r, W2_l, b2_l, W2_r):
    src = edge_index[0].astype(jnp.int32)
    dst = edge_index[1].astype(jnp.int32)
    npad = NCH_PAD * CH - E
    pad_i = jnp.arange(npad, dtype=jnp.int32)
    srcp = jnp.concatenate([src, (pad_i * 97) % N]).reshape(SST, SS, CH)
    dstp = jnp.concatenate([dst, N + pad_i % (NP - N)]).reshape(SST, SS, CH)

    agg1_f, cnt_f = _sc_layer1(x, srcp, dstp)
    agg1 = agg1_f.reshape(NC, NP, D_IN)
    cnt = cnt_f.reshape(NC, NP, D_IN)

    h = _tc_dense1(agg1, cnt, x, W1_l, b1_l.reshape(2, 1, 128), W1_r)

    agg2_f = _sc_layer2(h[0], h[1], srcp, dstp)
    agg2 = agg2_f.reshape(2, NC, NP, D_IN)

    return _tc_dense2(agg2, cnt, h, W2_l, b2_l.reshape(1, D_HID), W2_r)
